# baseline (device time: 11699 ns/iter reference)
import jax
import jax.numpy as jnp
from jax import lax
from jax.experimental import pallas as pl
from jax.experimental.pallas import tpu as pltpu

NCHUNK = 2


def kernel(dy, W):
    m, k = dy.shape
    d = W.shape[0]
    rows = m // NCHUNK

    dy = pltpu.with_memory_space_constraint(dy, pltpu.MemorySpace.HBM)
    W = pltpu.with_memory_space_constraint(W, pltpu.MemorySpace.HBM)

    def body(
        dy_hbm,
        w_hbm,
        out_ref,
        dyv,
        wv,
        q_ref,
        s_ref,
        in_sems,
        qsend,
        qrecv,
        ssend,
        srecv,
    ):
        my_x = lax.axis_index("x")
        my_y = lax.axis_index("y")
        my_z = lax.axis_index("z")
        peer = (1 - my_x, my_y, my_z)

        barrier = pltpu.get_barrier_semaphore()
        pl.semaphore_signal(
            barrier, inc=1, device_id=peer, device_id_type=pl.DeviceIdType.MESH
        )

        sl0 = pl.ds(0, rows)
        sl1 = pl.ds(rows, m - rows)
        w_dma = pltpu.make_async_copy(w_hbm, wv, in_sems.at[1])
        dy0_dma = pltpu.make_async_copy(dy_hbm.at[sl0, :], dyv.at[sl0, :],
                                        in_sems.at[0])
        w_dma.start()
        dy0_dma.start()
        dy0_dma.wait()
        w_dma.wait()
        dy1_dma = pltpu.make_async_copy(dy_hbm.at[sl1, :], dyv.at[sl1, :],
                                        in_sems.at[0])
        dy1_dma.start()

        rdmas = []
        partials = []
        for c in range(NCHUNK):
            sl = pl.ds(c * rows, rows)
            if c == 1:
                dy1_dma.wait()
            partial = lax.dot_general(
                dyv[sl, :],
                wv[...],
                (((1,), (1,)), ((), ())),
                preferred_element_type=jnp.float32,
            )
            scale = jnp.max(jnp.abs(partial)) / 127.0 + 1e-30
            q_ref[0, sl, :] = jnp.rint(partial * (1.0 / scale)).astype(jnp.int8)
            s_ref[0, c] = jnp.full((8, 128), scale, jnp.float32)
            if c == 0:
                pl.semaphore_wait(barrier, 1)
            qr = pltpu.make_async_remote_copy(
                src_ref=q_ref.at[0, sl, :],
                dst_ref=q_ref.at[1, sl, :],
                send_sem=qsend.at[c],
                recv_sem=qrecv.at[c],
                device_id=peer,
                device_id_type=pl.DeviceIdType.MESH,
            )
            qr.start()
            sr = pltpu.make_async_remote_copy(
                src_ref=s_ref.at[0, c],
                dst_ref=s_ref.at[1, c],
                send_sem=ssend.at[c],
                recv_sem=srecv.at[c],
                device_id=peer,
                device_id_type=pl.DeviceIdType.MESH,
            )
            sr.start()
            rdmas.append((qr, sr))
            partials.append(partial)

        for c in range(NCHUNK):
            sl = pl.ds(c * rows, rows)
            qr, sr = rdmas[c]
            qr.wait()
            sr.wait()
            peer_scale = s_ref[1, c, 0:1, 0:1]
            out_ref[sl, :] = partials[c] + q_ref[1, sl, :].astype(
                jnp.float32
            ) * peer_scale

    return pl.pallas_call(
        body,
        out_shape=jax.ShapeDtypeStruct((m, d), jnp.float32),
        in_specs=[
            pl.BlockSpec(memory_space=pl.ANY),
            pl.BlockSpec(memory_space=pl.ANY),
        ],
        out_specs=pl.BlockSpec(memory_space=pltpu.VMEM),
        scratch_shapes=[
            pltpu.VMEM((m, k), jnp.float32),
            pltpu.VMEM((d, k), jnp.float32),
            pltpu.VMEM((2, m, d), jnp.int8),
            pltpu.VMEM((2, NCHUNK, 8, 128), jnp.float32),
            pltpu.SemaphoreType.DMA((2,)),
            pltpu.SemaphoreType.DMA((NCHUNK,)),
            pltpu.SemaphoreType.DMA((NCHUNK,)),
            pltpu.SemaphoreType.DMA((NCHUNK,)),
            pltpu.SemaphoreType.DMA((NCHUNK,)),
        ],
        compiler_params=pltpu.CompilerParams(collective_id=0),
    )(dy, W)
